# baseline (speedup 1.0000x reference)
import jax
import jax.numpy as jnp
from jax import lax
from jax.experimental import pallas as pl
from jax.experimental.pallas import tpu as pltpu

_N_E = 8192
_D = 32
_BETA = 0.25
_RB = 512
_CB = 512
_NCB = _N_E // _CB


def _argmin_body(z_ref, zs_ref, w_ref, ws_ref, idx_ref, loss_ref, run_d, run_i, acc):
    j = pl.program_id(1)
    z = z_ref[...]
    w = w_ref[...]
    dot = lax.dot_general(z, w, (((1,), (1,)), ((), ())),
                          preferred_element_type=jnp.float32)
    sq = (zs_ref[...] + ws_ref[...]) - 2.0 * dot
    d = jnp.sqrt(jnp.maximum(sq, 0.0))
    bmin = jnp.min(d, axis=1, keepdims=True)
    ii = lax.broadcasted_iota(jnp.int32, (_RB, _CB), 1) + j * _CB
    cand = jnp.where(d == bmin, ii, jnp.int32(2**31 - 1))
    bidx = jnp.min(cand, axis=1, keepdims=True)

    @pl.when(j == 0)
    def _():
        run_d[...] = bmin
        run_i[...] = bidx

    @pl.when(j > 0)
    def _():
        upd = bmin < run_d[...]
        run_d[...] = jnp.where(upd, bmin, run_d[...])
        run_i[...] = jnp.where(upd, bidx, run_i[...])

    @pl.when(j == _NCB - 1)
    def _():
        i = pl.program_id(0)
        idx_ref[...] = run_i[...]
        rd = run_d[...]
        part = jnp.sum(rd * rd)

        @pl.when(i == 0)
        def _():
            acc[0, 0] = part

        @pl.when(i > 0)
        def _():
            acc[0, 0] = acc[0, 0] + part

        @pl.when(i == pl.num_programs(0) - 1)
        def _():
            m = acc[0, 0] / (pl.num_programs(0) * _RB * _D)
            loss_ref[...] = jnp.reshape(m + _BETA * m, (1, 1))


def _gather_body(z_ref, w_ref, idx_ref, zq_ref):
    z = z_ref[...]
    run_i = idx_ref[...]

    def body(j, zq):
        w = w_ref[pl.ds(j * _CB, _CB), :]
        ii = lax.broadcasted_iota(jnp.int32, (_RB, _CB), 1) + j * _CB
        oh = (run_i == ii).astype(jnp.float32)
        return zq + lax.dot_general(oh, w, (((1,), (0,)), ((), ())),
                                    preferred_element_type=jnp.float32,
                                    precision=lax.Precision.HIGHEST)

    zq = lax.fori_loop(0, _NCB, body, jnp.zeros((_RB, _D), jnp.float32))
    zq_ref[...] = z + (zq - z)


def kernel(z, W):
    zf = z.reshape(-1, _D)
    r = zf.shape[0]
    nrb = r // _RB
    zsum = jnp.sum(zf ** 2, axis=1, keepdims=True)
    wsum = jnp.sum(W ** 2, axis=1)[None, :]
    idx, loss = pl.pallas_call(
        _argmin_body,
        grid=(nrb, _NCB),
        in_specs=[
            pl.BlockSpec((_RB, _D), lambda i, j: (i, 0)),
            pl.BlockSpec((_RB, 1), lambda i, j: (i, 0)),
            pl.BlockSpec((_CB, _D), lambda i, j: (j, 0)),
            pl.BlockSpec((1, _CB), lambda i, j: (0, j)),
        ],
        out_specs=[
            pl.BlockSpec((_RB, 1), lambda i, j: (i, 0)),
            pl.BlockSpec((1, 1), lambda i, j: (0, 0)),
        ],
        out_shape=[
            jax.ShapeDtypeStruct((r, 1), jnp.int32),
            jax.ShapeDtypeStruct((1, 1), jnp.float32),
        ],
        scratch_shapes=[
            pltpu.VMEM((_RB, 1), jnp.float32),
            pltpu.VMEM((_RB, 1), jnp.int32),
            pltpu.SMEM((1, 1), jnp.float32),
        ],
    )(zf, zsum, W, wsum)

    zq = pl.pallas_call(
        _gather_body,
        grid=(nrb,),
        in_specs=[
            pl.BlockSpec((_RB, _D), lambda i: (i, 0)),
            pl.BlockSpec((_N_E, _D), lambda i: (0, 0)),
            pl.BlockSpec((_RB, 1), lambda i: (i, 0)),
        ],
        out_specs=pl.BlockSpec((_RB, _D), lambda i: (i, 0)),
        out_shape=jax.ShapeDtypeStruct((r, _D), jnp.float32),
    )(zf, W, idx)

    return zq.reshape(z.shape), loss.reshape(()), idx.reshape(r)


# speedup vs baseline: 1.6991x; 1.6991x over previous
import functools

import jax
import jax.numpy as jnp
from jax import lax
from jax.experimental import pallas as pl
from jax.experimental.pallas import tpu as pltpu
from jax.experimental.pallas import tpu_sc as plsc

_N_E = 8192
_D = 32
_BETA = 0.25
_RB = 512
_CB = 512
_NCB = _N_E // _CB

_NW = 32
_R = 4608
_BPW = _R // _NW
_HALF = _BPW // 2


def _argmin_body(z_ref, zs_ref, w_ref, ws_ref, idx_ref, loss_ref, run_d, run_i, acc):
    j = pl.program_id(1)
    z = z_ref[...]
    w = w_ref[...]
    dot = lax.dot_general(z, w, (((1,), (1,)), ((), ())),
                          preferred_element_type=jnp.float32)
    sq = (zs_ref[...] + ws_ref[...]) - 2.0 * dot
    d = jnp.sqrt(jnp.maximum(sq, 0.0))
    bmin = jnp.min(d, axis=1, keepdims=True)
    ii = lax.broadcasted_iota(jnp.int32, (_RB, _CB), 1) + j * _CB
    cand = jnp.where(d == bmin, ii, jnp.int32(2**31 - 1))
    bidx = jnp.min(cand, axis=1, keepdims=True)

    @pl.when(j == 0)
    def _():
        run_d[...] = bmin
        run_i[...] = bidx

    @pl.when(j > 0)
    def _():
        upd = bmin < run_d[...]
        run_d[...] = jnp.where(upd, bmin, run_d[...])
        run_i[...] = jnp.where(upd, bidx, run_i[...])

    @pl.when(j == _NCB - 1)
    def _():
        i = pl.program_id(0)
        idx_ref[...] = run_i[...]
        rd = run_d[...]
        part = jnp.sum(rd * rd)

        @pl.when(i == 0)
        def _():
            acc[0, 0] = part

        @pl.when(i > 0)
        def _():
            acc[0, 0] = acc[0, 0] + part

        @pl.when(i == pl.num_programs(0) - 1)
        def _():
            m = acc[0, 0] / (pl.num_programs(0) * _RB * _D)
            loss_ref[...] = jnp.reshape(m + _BETA * m, (1, 1))


def _sc_gather(W, idx):
    mesh = plsc.VectorSubcoreMesh(core_axis_name="c", subcore_axis_name="s")

    @functools.partial(
        pl.kernel,
        mesh=mesh,
        out_type=jax.ShapeDtypeStruct((_R, _D), jnp.float32),
        scratch_types=[
            pltpu.VMEM((_HALF,), jnp.int32),
            pltpu.VMEM((_HALF,), jnp.int32),
            pltpu.VMEM((_HALF, _D), jnp.float32),
            pltpu.VMEM((_HALF, _D), jnp.float32),
            pltpu.SemaphoreType.DMA,
        ],
        compiler_params=pltpu.CompilerParams(use_tc_tiling_on_sc=False),
    )
    def k(w_hbm, idx_hbm, out_hbm, idx_a, idx_b, rows_a, rows_b, sem):
        wid = lax.axis_index("s") * 2 + lax.axis_index("c")
        base = wid * _BPW
        pltpu.sync_copy(idx_hbm.at[pl.ds(base, _HALF)], idx_a)
        pltpu.sync_copy(idx_hbm.at[pl.ds(base + _HALF, _HALF)], idx_b)
        ca = pltpu.async_copy(w_hbm.at[idx_a], rows_a, sem)
        cb = pltpu.async_copy(w_hbm.at[idx_b], rows_b, sem)
        ca.wait()
        cb.wait()
        pltpu.sync_copy(rows_a, out_hbm.at[pl.ds(base, _HALF)])
        pltpu.sync_copy(rows_b, out_hbm.at[pl.ds(base + _HALF, _HALF)])

    return k(W, idx)


def kernel(z, W):
    zf = z.reshape(-1, _D)
    r = zf.shape[0]
    nrb = r // _RB
    zsum = jnp.sum(zf ** 2, axis=1, keepdims=True)
    wsum = jnp.sum(W ** 2, axis=1)[None, :]
    idx, loss = pl.pallas_call(
        _argmin_body,
        grid=(nrb, _NCB),
        in_specs=[
            pl.BlockSpec((_RB, _D), lambda i, j: (i, 0)),
            pl.BlockSpec((_RB, 1), lambda i, j: (i, 0)),
            pl.BlockSpec((_CB, _D), lambda i, j: (j, 0)),
            pl.BlockSpec((1, _CB), lambda i, j: (0, j)),
        ],
        out_specs=[
            pl.BlockSpec((_RB, 1), lambda i, j: (i, 0)),
            pl.BlockSpec((1, 1), lambda i, j: (0, 0)),
        ],
        out_shape=[
            jax.ShapeDtypeStruct((r, 1), jnp.int32),
            jax.ShapeDtypeStruct((1, 1), jnp.float32),
        ],
        scratch_shapes=[
            pltpu.VMEM((_RB, 1), jnp.float32),
            pltpu.VMEM((_RB, 1), jnp.int32),
            pltpu.SMEM((1, 1), jnp.float32),
        ],
    )(zf, zsum, W, wsum)

    idx_flat = idx.reshape(r)
    zq = _sc_gather(W, idx_flat)
    return zq.reshape(z.shape), loss.reshape(()), idx_flat


# speedup vs baseline: 1.7113x; 1.0072x over previous
import functools

import jax
import jax.numpy as jnp
from jax import lax
from jax.experimental import pallas as pl
from jax.experimental.pallas import tpu as pltpu
from jax.experimental.pallas import tpu_sc as plsc

_N_E = 8192
_D = 32
_BETA = 0.25
_RB = 512
_CB = 512
_NCB = _N_E // _CB

_NW = 32
_R = 4608
_BPW = _R // _NW
_HALF = _BPW // 2


def _argmin_body(z_ref, zs_ref, w2_ref, ws_ref, idx_ref, loss_ref, run_d, run_i, acc):
    j = pl.program_id(1)
    z = z_ref[...]
    w2 = w2_ref[...]
    dot2 = lax.dot_general(z, w2, (((1,), (1,)), ((), ())),
                           preferred_element_type=jnp.float32)
    sq = (zs_ref[...] + ws_ref[...]) - dot2
    d = jnp.sqrt(jnp.maximum(sq, 0.0))
    bmin = jnp.min(d, axis=1, keepdims=True)
    ii = lax.broadcasted_iota(jnp.int32, (_RB, _CB), 1)
    cand = jnp.where(d == bmin, ii, jnp.int32(1 << 30))
    bidx = jnp.min(cand, axis=1, keepdims=True) + j * _CB

    @pl.when(j == 0)
    def _():
        run_d[...] = bmin
        run_i[...] = bidx

    @pl.when(j > 0)
    def _():
        upd = bmin < run_d[...]
        run_d[...] = jnp.where(upd, bmin, run_d[...])
        run_i[...] = jnp.where(upd, bidx, run_i[...])

    @pl.when(j == _NCB - 1)
    def _():
        i = pl.program_id(0)
        idx_ref[...] = run_i[...]
        rd = run_d[...]
        part = jnp.sum(rd * rd)

        @pl.when(i == 0)
        def _():
            acc[0, 0] = part

        @pl.when(i > 0)
        def _():
            acc[0, 0] = acc[0, 0] + part

        @pl.when(i == pl.num_programs(0) - 1)
        def _():
            m = acc[0, 0] / (pl.num_programs(0) * _RB * _D)
            loss_ref[...] = jnp.reshape(m + _BETA * m, (1, 1))


def _sc_gather(W, idx):
    mesh = plsc.VectorSubcoreMesh(core_axis_name="c", subcore_axis_name="s")

    @functools.partial(
        pl.kernel,
        mesh=mesh,
        out_type=jax.ShapeDtypeStruct((_R, _D), jnp.float32),
        scratch_types=[
            pltpu.VMEM((_HALF,), jnp.int32),
            pltpu.VMEM((_HALF,), jnp.int32),
            pltpu.VMEM((_HALF, _D), jnp.float32),
            pltpu.VMEM((_HALF, _D), jnp.float32),
            pltpu.SemaphoreType.DMA,
        ],
        compiler_params=pltpu.CompilerParams(use_tc_tiling_on_sc=False),
    )
    def k(w_hbm, idx_hbm, out_hbm, idx_a, idx_b, rows_a, rows_b, sem):
        wid = lax.axis_index("s") * 2 + lax.axis_index("c")
        base = wid * _BPW
        pltpu.sync_copy(idx_hbm.at[pl.ds(base, _HALF)], idx_a)
        pltpu.sync_copy(idx_hbm.at[pl.ds(base + _HALF, _HALF)], idx_b)
        ca = pltpu.async_copy(w_hbm.at[idx_a], rows_a, sem)
        cb = pltpu.async_copy(w_hbm.at[idx_b], rows_b, sem)
        ca.wait()
        cb.wait()
        pltpu.sync_copy(rows_a, out_hbm.at[pl.ds(base, _HALF)])
        pltpu.sync_copy(rows_b, out_hbm.at[pl.ds(base + _HALF, _HALF)])

    return k(W, idx)


def kernel(z, W):
    zf = z.reshape(-1, _D)
    r = zf.shape[0]
    nrb = r // _RB
    zsum = jnp.sum(zf ** 2, axis=1, keepdims=True)
    wsum = jnp.sum(W ** 2, axis=1)[None, :]
    idx, loss = pl.pallas_call(
        _argmin_body,
        grid=(nrb, _NCB),
        in_specs=[
            pl.BlockSpec((_RB, _D), lambda i, j: (i, 0)),
            pl.BlockSpec((_RB, 1), lambda i, j: (i, 0)),
            pl.BlockSpec((_CB, _D), lambda i, j: (j, 0)),
            pl.BlockSpec((1, _CB), lambda i, j: (0, j)),
        ],
        out_specs=[
            pl.BlockSpec((_RB, 1), lambda i, j: (i, 0)),
            pl.BlockSpec((1, 1), lambda i, j: (0, 0)),
        ],
        out_shape=[
            jax.ShapeDtypeStruct((r, 1), jnp.int32),
            jax.ShapeDtypeStruct((1, 1), jnp.float32),
        ],
        scratch_shapes=[
            pltpu.VMEM((_RB, 1), jnp.float32),
            pltpu.VMEM((_RB, 1), jnp.int32),
            pltpu.SMEM((1, 1), jnp.float32),
        ],
    )(zf, zsum, W + W, wsum)

    idx_flat = idx.reshape(r)
    zq = _sc_gather(W, idx_flat)
    return zq.reshape(z.shape), loss.reshape(()), idx_flat


# speedup vs baseline: 2.4587x; 1.4367x over previous
import functools

import jax
import jax.numpy as jnp
from jax import lax
from jax.experimental import pallas as pl
from jax.experimental.pallas import tpu as pltpu
from jax.experimental.pallas import tpu_sc as plsc

_N_E = 8192
_D = 32
_BETA = 0.25
_RB = 512
_CB = 8192
_NCB = _N_E // _CB

_NW = 32
_R = 4608
_BPW = _R // _NW
_HALF = _BPW // 2


def _argmin_body(z_ref, zs_ref, w2_ref, ws_ref, idx_ref, loss_ref, run_d, run_i, acc):
    j = pl.program_id(1)
    z = z_ref[...]
    dot2 = lax.dot_general(z, w2_ref[...], (((1,), (1,)), ((), ())),
                           preferred_element_type=jnp.float32)
    sq = (zs_ref[...] + ws_ref[...]) - dot2
    d = jnp.sqrt(jnp.maximum(sq, 0.0))
    bmin = jnp.min(d, axis=1, keepdims=True)
    ii = lax.broadcasted_iota(jnp.int32, (_RB, _CB), 1)
    cand = jnp.where(d == bmin, ii, jnp.int32(1 << 30))
    bidx = jnp.min(cand, axis=1, keepdims=True) + j * _CB

    @pl.when(j == 0)
    def _():
        run_d[...] = bmin
        run_i[...] = bidx

    @pl.when(j > 0)
    def _():
        upd = bmin < run_d[...]
        run_d[...] = jnp.where(upd, bmin, run_d[...])
        run_i[...] = jnp.where(upd, bidx, run_i[...])

    @pl.when(j == _NCB - 1)
    def _():
        i = pl.program_id(0)
        idx_ref[...] = run_i[...]
        rd = run_d[...]
        part = jnp.sum(rd * rd)

        @pl.when(i == 0)
        def _():
            acc[0, 0] = part

        @pl.when(i > 0)
        def _():
            acc[0, 0] = acc[0, 0] + part

        @pl.when(i == pl.num_programs(0) - 1)
        def _():
            m = acc[0, 0] / (pl.num_programs(0) * _RB * _D)
            loss_ref[...] = jnp.reshape(m + _BETA * m, (1, 1))


def _sc_gather(W, idx):
    mesh = plsc.VectorSubcoreMesh(core_axis_name="c", subcore_axis_name="s")

    @functools.partial(
        pl.kernel,
        mesh=mesh,
        out_type=jax.ShapeDtypeStruct((_R, _D), jnp.float32),
        scratch_types=[
            pltpu.VMEM((_HALF,), jnp.int32),
            pltpu.VMEM((_HALF,), jnp.int32),
            pltpu.VMEM((_HALF, _D), jnp.float32),
            pltpu.VMEM((_HALF, _D), jnp.float32),
            pltpu.SemaphoreType.DMA,
        ],
        compiler_params=pltpu.CompilerParams(use_tc_tiling_on_sc=False),
    )
    def k(w_hbm, idx_hbm, out_hbm, idx_a, idx_b, rows_a, rows_b, sem):
        wid = lax.axis_index("s") * 2 + lax.axis_index("c")
        base = wid * _BPW
        pltpu.sync_copy(idx_hbm.at[pl.ds(base, _HALF)], idx_a)
        pltpu.sync_copy(idx_hbm.at[pl.ds(base + _HALF, _HALF)], idx_b)
        ca = pltpu.async_copy(w_hbm.at[idx_a], rows_a, sem)
        cb = pltpu.async_copy(w_hbm.at[idx_b], rows_b, sem)
        ca.wait()
        cb.wait()
        pltpu.sync_copy(rows_a, out_hbm.at[pl.ds(base, _HALF)])
        pltpu.sync_copy(rows_b, out_hbm.at[pl.ds(base + _HALF, _HALF)])

    return k(W, idx)


def kernel(z, W):
    zf = z.reshape(-1, _D)
    r = zf.shape[0]
    nrb = r // _RB
    zsum = jnp.sum(zf ** 2, axis=1, keepdims=True)
    wsum = jnp.sum(W ** 2, axis=1)[None, :]
    idx, loss = pl.pallas_call(
        _argmin_body,
        grid=(nrb, _NCB),
        in_specs=[
            pl.BlockSpec((_RB, _D), lambda i, j: (i, 0)),
            pl.BlockSpec((_RB, 1), lambda i, j: (i, 0)),
            pl.BlockSpec((_CB, _D), lambda i, j: (j, 0)),
            pl.BlockSpec((1, _CB), lambda i, j: (0, j)),
        ],
        out_specs=[
            pl.BlockSpec((_RB, 1), lambda i, j: (i, 0)),
            pl.BlockSpec((1, 1), lambda i, j: (0, 0)),
        ],
        out_shape=[
            jax.ShapeDtypeStruct((r, 1), jnp.int32),
            jax.ShapeDtypeStruct((1, 1), jnp.float32),
        ],
        scratch_shapes=[
            pltpu.VMEM((_RB, 1), jnp.float32),
            pltpu.VMEM((_RB, 1), jnp.int32),
            pltpu.SMEM((1, 1), jnp.float32),
        ],
    )(zf, zsum, W + W, wsum)

    idx_flat = idx.reshape(r)
    zq = _sc_gather(W, idx_flat)
    return zq.reshape(z.shape), loss.reshape(()), idx_flat
